# baseline (speedup 1.0000x reference)
import functools

import jax
import jax.numpy as jnp
from jax import lax
from jax.experimental import pallas as pl
from jax.experimental.pallas import tpu as pltpu
from jax.experimental.pallas import tpu_sc as plsc

_B, _H, _W, _C = 4, 384, 384, 96
_HW = _H * _W
_NW = 32
_ROWS_PER_W = _B * _H // _NW
_WORKERS_PER_B = _NW // _B
_CPR = _W // 64
_NT = _ROWS_PER_W * _CPR
_STEP = 2.0 / (_W - 1.0)
_SCALE = (_W - 1.0) / 2.0


def _bf16_round(v):
    u = plsc.bitcast(v, jnp.int32)
    rnd = lax.shift_right_logical(u, jnp.int32(16)) & jnp.int32(1)
    u = (u + jnp.int32(0x7FFF) + rnd) & jnp.int32(-65536)
    return plsc.bitcast(u, jnp.float32)


def _body(table_hbm, cos_hbm, sin_hbm, out_hbm,
          idx_v, rows_v, w_v, obuf_v, cos_v, sin_v,
          gsem0, gsem1, osem0, osem1):
    wid = lax.axis_index("s") * 2 + lax.axis_index("c")
    pltpu.sync_copy(cos_hbm.at[pl.ds(wid * 16, 16)], cos_v)
    pltpu.sync_copy(sin_hbm.at[pl.ds(wid * 16, 16)], sin_v)
    cos = _bf16_round(cos_v[...])
    sin = _bf16_round(sin_v[...])
    b = wid // _WORKERS_PER_B
    i0 = (wid % _WORKERS_PER_B) * _ROWS_PER_W
    bbase = b * _HW
    lane = lax.iota(jnp.int32, 16)
    gsems = (gsem0, gsem1)
    osems = (osem0, osem1)

    def compute_issue(t, buf):
        i = i0 + t // _CPR
        jb = (t % _CPR) * 64
        gy = _bf16_round(
            jnp.full((16,), i, jnp.int32).astype(jnp.float32) * _STEP - 1.0)
        for k in range(4):
            j = jnp.full((16,), jb + k * 16, jnp.int32) + lane
            gx = _bf16_round(j.astype(jnp.float32) * _STEP - 1.0)
            x = (cos * gx - sin * gy + 1.0) * _SCALE
            y = (sin * gx + cos * gy + 1.0) * _SCALE
            xt = x.astype(jnp.int32)
            yt = y.astype(jnp.int32)
            xf = xt - jnp.where(xt.astype(jnp.float32) > x, 1, 0)
            yf = yt - jnp.where(yt.astype(jnp.float32) > y, 1, 0)
            x0 = jnp.clip(xf, 0, _W - 1)
            x1 = jnp.clip(xf + 1, 0, _W - 1)
            y0 = jnp.clip(yf, 0, _H - 1)
            y1 = jnp.clip(yf + 1, 0, _H - 1)
            x0f = x0.astype(jnp.float32)
            x1f = x1.astype(jnp.float32)
            y0f = y0.astype(jnp.float32)
            y1f = y1.astype(jnp.float32)
            w_v[buf, k * 4 + 0, :] = (x1f - x) * (y1f - y)
            w_v[buf, k * 4 + 1, :] = (x1f - x) * (y - y0f)
            w_v[buf, k * 4 + 2, :] = (x - x0f) * (y1f - y)
            w_v[buf, k * 4 + 3, :] = (x - x0f) * (y - y0f)
            ra = bbase + y0 * _W
            rb = bbase + y1 * _W
            h, kk = divmod(k, 2)
            base = kk * 64
            idx_v[buf, h, pl.ds(base, 16)] = ra + x0
            idx_v[buf, h, pl.ds(base + 16, 16)] = rb + x0
            idx_v[buf, h, pl.ds(base + 32, 16)] = ra + x1
            idx_v[buf, h, pl.ds(base + 48, 16)] = rb + x1
        for h in range(2):
            pltpu.async_copy(table_hbm.at[idx_v.at[buf, h]],
                             rows_v.at[buf, pl.ds(h * 128, 128)],
                             gsems[buf])

    def gather_wait(buf):
        for h in range(2):
            pltpu.make_async_copy(table_hbm.at[idx_v.at[buf, h]],
                                  rows_v.at[buf, pl.ds(h * 128, 128)],
                                  gsems[buf]).wait()

    def out_wait(buf):
        pltpu.make_async_copy(obuf_v.at[buf], out_hbm.at[pl.ds(bbase, 64)],
                              osems[buf]).wait()

    def combine(buf):
        for k in range(4):
            base = k * 64
            wva = w_v[buf, k * 4 + 0, :]
            wvb = w_v[buf, k * 4 + 1, :]
            wvc = w_v[buf, k * 4 + 2, :]
            wvd = w_v[buf, k * 4 + 3, :]
            for p in range(16):
                wav = jnp.full((16,), wva[p])
                wbv = jnp.full((16,), wvb[p])
                wcv = jnp.full((16,), wvc[p])
                wdv = jnp.full((16,), wvd[p])
                for cg in range(_C // 16):
                    sl = pl.ds(cg * 16, 16)
                    t0 = wav * rows_v[buf, base + p, sl]
                    t1 = wbv * rows_v[buf, base + 16 + p, sl]
                    t2 = wcv * rows_v[buf, base + 32 + p, sl]
                    t3 = wdv * rows_v[buf, base + 48 + p, sl]
                    obuf_v[buf, k * 16 + p, sl] = (t0 + t1) + (t2 + t3)

    def out_issue(t, buf):
        i = i0 + t // _CPR
        jb = (t % _CPR) * 64
        pltpu.async_copy(obuf_v.at[buf],
                         out_hbm.at[pl.ds(bbase + i * _W + jb, 64)],
                         osems[buf])

    compute_issue(0, 0)

    @pl.loop(0, _NT, step=2)
    def _(tb):
        for buf in (0, 1):
            t = tb + buf

            @pl.when(t + 1 < _NT)
            def _():
                compute_issue(t + 1, buf ^ 1)

            gather_wait(buf)

            @pl.when(t >= 2)
            def _():
                out_wait(buf)

            combine(buf)
            out_issue(t, buf)

    out_wait(0)
    out_wait(1)


@functools.partial(
    pl.kernel,
    out_type=jax.ShapeDtypeStruct((_B * _HW, _C), jnp.float32),
    mesh=plsc.VectorSubcoreMesh(core_axis_name="c", subcore_axis_name="s"),
    scratch_types=[
        pltpu.VMEM((2, 2, 128), jnp.int32),
        pltpu.VMEM((2, 256, _C), jnp.float32),
        pltpu.VMEM((2, 16, 16), jnp.float32),
        pltpu.VMEM((2, 64, _C), jnp.float32),
        pltpu.VMEM((16,), jnp.float32),
        pltpu.VMEM((16,), jnp.float32),
        pltpu.SemaphoreType.DMA,
        pltpu.SemaphoreType.DMA,
        pltpu.SemaphoreType.DMA,
        pltpu.SemaphoreType.DMA,
    ],
    compiler_params=pltpu.CompilerParams(
        use_tc_tiling_on_sc=False, needs_layout_passes=False),
)
def _sample_kernel(table_hbm, cos_hbm, sin_hbm, out_hbm,
                   idx_v, rows_v, w_v, obuf_v, cos_v, sin_v,
                   gsem0, gsem1, osem0, osem1):
    _body(table_hbm, cos_hbm, sin_hbm, out_hbm,
          idx_v, rows_v, w_v, obuf_v, cos_v, sin_v,
          gsem0, gsem1, osem0, osem1)


def kernel(U, theta):
    table = U.reshape(_B * _HW, _C)
    t = theta[:, 0]
    cos_rep = jnp.repeat(jnp.cos(t), _WORKERS_PER_B * 16)
    sin_rep = jnp.repeat(jnp.sin(t), _WORKERS_PER_B * 16)
    out = _sample_kernel(table, cos_rep, sin_rep)
    return out.reshape(_B, _H, _W, _C)


# speedup vs baseline: 1.0208x; 1.0208x over previous
import functools

import jax
import jax.numpy as jnp
from jax import lax
from jax.experimental import pallas as pl
from jax.experimental.pallas import tpu as pltpu
from jax.experimental.pallas import tpu_sc as plsc

_B, _H, _W, _C = 4, 384, 384, 96
_HW = _H * _W
_NW = 32
_ROWS_PER_W = _B * _H // _NW
_WORKERS_PER_B = _NW // _B
_CPR = _W // 32
_NT = _ROWS_PER_W * _CPR
_STEP = 2.0 / (_W - 1.0)
_SCALE = (_W - 1.0) / 2.0


def _bf16_round(v):
    u = plsc.bitcast(v, jnp.int32)
    rnd = lax.shift_right_logical(u, jnp.int32(16)) & jnp.int32(1)
    u = (u + jnp.int32(0x7FFF) + rnd) & jnp.int32(-65536)
    return plsc.bitcast(u, jnp.float32)


def _body(table_hbm, cos_hbm, sin_hbm, out_hbm,
          idx_v, rows_v, w_v, obuf_v, cos_v, sin_v,
          gsem0, gsem1, osem0, osem1):
    wid = lax.axis_index("s") * 2 + lax.axis_index("c")
    pltpu.sync_copy(cos_hbm.at[pl.ds(wid * 16, 16)], cos_v)
    pltpu.sync_copy(sin_hbm.at[pl.ds(wid * 16, 16)], sin_v)
    cos = _bf16_round(cos_v[...])
    sin = _bf16_round(sin_v[...])
    b = wid // _WORKERS_PER_B
    i0 = (wid % _WORKERS_PER_B) * _ROWS_PER_W
    bbase = b * _HW
    lane = lax.iota(jnp.int32, 16)
    gsems = (gsem0, gsem1)
    osems = (osem0, osem1)

    def compute_issue(t, buf):
        i = i0 + t // _CPR
        jb = (t % _CPR) * 32
        gy = _bf16_round(
            jnp.full((16,), i, jnp.int32).astype(jnp.float32) * _STEP - 1.0)
        for k in range(2):
            j = jnp.full((16,), jb + k * 16, jnp.int32) + lane
            gx = _bf16_round(j.astype(jnp.float32) * _STEP - 1.0)
            x = (cos * gx - sin * gy + 1.0) * _SCALE
            y = (sin * gx + cos * gy + 1.0) * _SCALE
            xt = x.astype(jnp.int32)
            yt = y.astype(jnp.int32)
            xf = xt - jnp.where(xt.astype(jnp.float32) > x, 1, 0)
            yf = yt - jnp.where(yt.astype(jnp.float32) > y, 1, 0)
            x0 = jnp.clip(xf, 0, _W - 1)
            x1 = jnp.clip(xf + 1, 0, _W - 1)
            y0 = jnp.clip(yf, 0, _H - 1)
            y1 = jnp.clip(yf + 1, 0, _H - 1)
            x0f = x0.astype(jnp.float32)
            x1f = x1.astype(jnp.float32)
            y0f = y0.astype(jnp.float32)
            y1f = y1.astype(jnp.float32)
            w_v[buf, k * 4 + 0, :] = (x1f - x) * (y1f - y)
            w_v[buf, k * 4 + 1, :] = (x1f - x) * (y - y0f)
            w_v[buf, k * 4 + 2, :] = (x - x0f) * (y1f - y)
            w_v[buf, k * 4 + 3, :] = (x - x0f) * (y - y0f)
            ra = bbase + y0 * _W
            rb = bbase + y1 * _W
            base = k * 64
            idx_v[buf, pl.ds(base, 16)] = ra + x0
            idx_v[buf, pl.ds(base + 16, 16)] = rb + x0
            idx_v[buf, pl.ds(base + 32, 16)] = ra + x1
            idx_v[buf, pl.ds(base + 48, 16)] = rb + x1
        pltpu.async_copy(table_hbm.at[idx_v.at[buf]], rows_v.at[buf],
                         gsems[buf])

    def gather_wait(buf):
        pltpu.make_async_copy(table_hbm.at[idx_v.at[buf]], rows_v.at[buf],
                              gsems[buf]).wait()

    def out_wait(buf):
        pltpu.make_async_copy(obuf_v.at[buf], out_hbm.at[pl.ds(bbase, 32)],
                              osems[buf]).wait()

    def combine(buf):
        for k in range(2):
            base = k * 64
            wva = w_v[buf, k * 4 + 0, :]
            wvb = w_v[buf, k * 4 + 1, :]
            wvc = w_v[buf, k * 4 + 2, :]
            wvd = w_v[buf, k * 4 + 3, :]
            for p in range(16):
                wav = jnp.full((16,), wva[p])
                wbv = jnp.full((16,), wvb[p])
                wcv = jnp.full((16,), wvc[p])
                wdv = jnp.full((16,), wvd[p])
                for cg in range(_C // 16):
                    sl = pl.ds(cg * 16, 16)
                    acc = (wav * rows_v[buf, base + p, sl]
                           + wbv * rows_v[buf, base + 16 + p, sl]
                           + wcv * rows_v[buf, base + 32 + p, sl]
                           + wdv * rows_v[buf, base + 48 + p, sl])
                    obuf_v[buf, k * 16 + p, sl] = acc

    def out_issue(t, buf):
        i = i0 + t // _CPR
        jb = (t % _CPR) * 32
        pltpu.async_copy(obuf_v.at[buf],
                         out_hbm.at[pl.ds(bbase + i * _W + jb, 32)],
                         osems[buf])

    compute_issue(0, 0)

    @pl.loop(0, _NT, step=2)
    def _(tb):
        for buf in (0, 1):
            t = tb + buf

            @pl.when(t + 1 < _NT)
            def _():
                compute_issue(t + 1, buf ^ 1)

            gather_wait(buf)

            @pl.when(t >= 2)
            def _():
                out_wait(buf)

            combine(buf)
            out_issue(t, buf)

    out_wait(0)
    out_wait(1)


@functools.partial(
    pl.kernel,
    out_type=jax.ShapeDtypeStruct((_B * _HW, _C), jnp.float32),
    mesh=plsc.VectorSubcoreMesh(core_axis_name="c", subcore_axis_name="s"),
    scratch_types=[
        pltpu.VMEM((2, 128), jnp.int32),
        pltpu.VMEM((2, 128, _C), jnp.float32),
        pltpu.VMEM((2, 8, 16), jnp.float32),
        pltpu.VMEM((2, 32, _C), jnp.float32),
        pltpu.VMEM((16,), jnp.float32),
        pltpu.VMEM((16,), jnp.float32),
        pltpu.SemaphoreType.DMA,
        pltpu.SemaphoreType.DMA,
        pltpu.SemaphoreType.DMA,
        pltpu.SemaphoreType.DMA,
    ],
    compiler_params=pltpu.CompilerParams(
        use_tc_tiling_on_sc=False, needs_layout_passes=False),
)
def _sample_kernel(table_hbm, cos_hbm, sin_hbm, out_hbm,
                   idx_v, rows_v, w_v, obuf_v, cos_v, sin_v,
                   gsem0, gsem1, osem0, osem1):
    _body(table_hbm, cos_hbm, sin_hbm, out_hbm,
          idx_v, rows_v, w_v, obuf_v, cos_v, sin_v,
          gsem0, gsem1, osem0, osem1)


def kernel(U, theta):
    table = U.reshape(_B * _HW, _C)
    t = theta[:, 0]
    cos_rep = jnp.repeat(jnp.cos(t), _WORKERS_PER_B * 16)
    sin_rep = jnp.repeat(jnp.sin(t), _WORKERS_PER_B * 16)
    out = _sample_kernel(table, cos_rep, sin_rep)
    return out.reshape(_B, _H, _W, _C)
